# Initial kernel scaffold; baseline (speedup 1.0000x reference)
import jax
import jax.numpy as jnp
from jax.experimental import pallas as pl


def kernel(sent_list, table):
    raise NotImplementedError("write your pallas kernel here")


# speedup vs baseline: 6.3744x; 6.3744x over previous
import functools

import jax
import jax.numpy as jnp
from jax import lax
from jax.experimental import pallas as pl
from jax.experimental.pallas import tpu as pltpu
from jax.experimental.pallas import tpu_sc as plsc

_BLK = 128
_K = 16


@functools.cache
def _make_gather(n_total, d):
    info = plsc.get_sparse_core_info()
    nc, ns = info.num_cores, info.num_subcores
    nw = nc * ns
    blocks_total = n_total // _BLK
    blocks_w = blocks_total // nw
    groups = blocks_w // _K
    rows_per_group = _K * _BLK
    mesh = plsc.VectorSubcoreMesh(core_axis_name="c", subcore_axis_name="s")

    @functools.partial(
        pl.kernel,
        mesh=mesh,
        out_type=jax.ShapeDtypeStruct((n_total, d), jnp.float32),
        scratch_types=[
            pltpu.VMEM((_K, _BLK), jnp.int32),
            pltpu.VMEM((rows_per_group, d), jnp.float32),
            pltpu.SemaphoreType.DMA,
        ],
        compiler_params=pltpu.CompilerParams(use_tc_tiling_on_sc=False),
    )
    def gather_kernel(idx_hbm, table_hbm, out_hbm, idx_v, rows_v, sem):
        wid = lax.axis_index("s") * nc + lax.axis_index("c")
        blk0 = wid * blocks_w

        def group(g, carry):
            b0 = blk0 + g * _K
            pltpu.sync_copy(idx_hbm.at[pl.ds(b0, _K)], idx_v)
            copies = [
                pltpu.async_copy(
                    table_hbm.at[idx_v.at[j]],
                    rows_v.at[pl.ds(j * _BLK, _BLK)],
                    sem,
                )
                for j in range(_K)
            ]
            for c in copies:
                c.wait()
            pltpu.sync_copy(rows_v, out_hbm.at[pl.ds(b0 * _BLK, rows_per_group)])
            return carry

        lax.fori_loop(0, groups, group, 0)

    return gather_kernel


def kernel(sent_list, table):
    b, l = sent_list.shape
    n_total = b * l
    d = table.shape[1]
    idx2d = sent_list.reshape(n_total // _BLK, _BLK).astype(jnp.int32)
    out = _make_gather(n_total, d)(idx2d, table.astype(jnp.float32))
    return out.reshape(b, l, d)


# speedup vs baseline: 6.4881x; 1.0178x over previous
import functools

import jax
import jax.numpy as jnp
from jax import lax
from jax.experimental import pallas as pl
from jax.experimental.pallas import tpu as pltpu
from jax.experimental.pallas import tpu_sc as plsc

_BLK = 128
_K = 8
_NBUF = 2


@functools.cache
def _make_gather(n_total, d):
    info = plsc.get_sparse_core_info()
    nc, ns = info.num_cores, info.num_subcores
    nw = nc * ns
    blocks_total = n_total // _BLK
    blocks_w = blocks_total // nw
    groups = blocks_w // _K
    assert groups % _NBUF == 0
    rows_per_group = _K * _BLK
    mesh = plsc.VectorSubcoreMesh(core_axis_name="c", subcore_axis_name="s")

    @functools.partial(
        pl.kernel,
        mesh=mesh,
        out_type=jax.ShapeDtypeStruct((n_total, d), jnp.float32),
        scratch_types=[
            pltpu.VMEM((_K, _BLK), jnp.int32),
            pltpu.VMEM((_K, _BLK), jnp.int32),
            pltpu.VMEM((rows_per_group, d), jnp.float32),
            pltpu.VMEM((rows_per_group, d), jnp.float32),
            pltpu.SemaphoreType.DMA,
            pltpu.SemaphoreType.DMA,
            pltpu.SemaphoreType.DMA,
            pltpu.SemaphoreType.DMA,
            pltpu.SemaphoreType.DMA,
        ],
        compiler_params=pltpu.CompilerParams(use_tc_tiling_on_sc=False),
    )
    def gather_kernel(idx_hbm, table_hbm, out_hbm, idx0, idx1, rows0, rows1,
                      sem_i0, sem_i1, sem_g, sem_o0, sem_o1):
        idx_v = (idx0, idx1)
        rows_v = (rows0, rows1)
        sem_i = (sem_i0, sem_i1)
        sem_o = (sem_o0, sem_o1)
        wid = lax.axis_index("s") * nc + lax.axis_index("c")
        blk0 = wid * blocks_w

        def idx_src(g):
            return idx_hbm.at[pl.ds(blk0 + g * _K, _K)]

        def do_group(g, b, wait_out):
            if wait_out:
                pltpu.make_async_copy(
                    rows_v[b], out_hbm.at[pl.ds(0, rows_per_group)], sem_o[b]
                ).wait()
            pltpu.make_async_copy(idx_src(0), idx_v[b], sem_i[b]).wait()
            copies = [
                pltpu.async_copy(
                    table_hbm.at[idx_v[b].at[j]],
                    rows_v[b].at[pl.ds(j * _BLK, _BLK)],
                    sem_g,
                )
                for j in range(_K)
            ]
            for c in copies:
                c.wait()
            gn = jnp.minimum(g + _NBUF, groups - 1)
            pltpu.async_copy(idx_src(gn), idx_v[b], sem_i[b])
            pltpu.async_copy(
                rows_v[b],
                out_hbm.at[pl.ds((blk0 + g * _K) * _BLK, rows_per_group)],
                sem_o[b],
            )

        for b in range(_NBUF):
            pltpu.async_copy(idx_src(b), idx_v[b], sem_i[b])
        for b in range(_NBUF):
            do_group(jnp.int32(b), b, wait_out=False)

        def pair(p, carry):
            for b in range(_NBUF):
                do_group(p * _NBUF + b, b, wait_out=True)
            return carry

        lax.fori_loop(1, groups // _NBUF, pair, 0)

        for b in range(_NBUF):
            pltpu.make_async_copy(idx_src(0), idx_v[b], sem_i[b]).wait()
            pltpu.make_async_copy(
                rows_v[b], out_hbm.at[pl.ds(0, rows_per_group)], sem_o[b]
            ).wait()

    return gather_kernel


def kernel(sent_list, table):
    b, l = sent_list.shape
    n_total = b * l
    d = table.shape[1]
    idx2d = sent_list.reshape(n_total // _BLK, _BLK).astype(jnp.int32)
    out = _make_gather(n_total, d)(idx2d, table.astype(jnp.float32))
    return out.reshape(b, l, d)


# speedup vs baseline: 6.9559x; 1.0721x over previous
import functools

import jax
import jax.numpy as jnp
from jax import lax
from jax.experimental import pallas as pl
from jax.experimental.pallas import tpu as pltpu
from jax.experimental.pallas import tpu_sc as plsc

_BLK = 128
_K = 5
_NBUF = 2


@functools.cache
def _make_gather(n_total, num_emb, d):
    info = plsc.get_sparse_core_info()
    nc, ns = info.num_cores, info.num_subcores
    nw = nc * ns
    blocks_total = n_total // _BLK
    blocks_w = blocks_total // nw
    groups = blocks_w // _K
    assert groups % _NBUF == 0
    rows_per_group = _K * _BLK
    stripe = -(-num_emb // ns)
    mesh = plsc.VectorSubcoreMesh(core_axis_name="c", subcore_axis_name="s")

    @functools.partial(
        pl.kernel,
        mesh=mesh,
        out_type=jax.ShapeDtypeStruct((n_total, d), jnp.float32),
        scratch_types=[
            pltpu.VMEM_SHARED((num_emb, d), jnp.float32),
            pltpu.VMEM((_K, _BLK), jnp.int32),
            pltpu.VMEM((_K, _BLK), jnp.int32),
            pltpu.VMEM((rows_per_group, d), jnp.float32),
            pltpu.VMEM((rows_per_group, d), jnp.float32),
            pltpu.SemaphoreType.DMA,
            pltpu.SemaphoreType.DMA,
            pltpu.SemaphoreType.DMA,
            pltpu.SemaphoreType.DMA,
            pltpu.SemaphoreType.DMA,
        ],
        compiler_params=pltpu.CompilerParams(use_tc_tiling_on_sc=False),
    )
    def gather_kernel(idx_hbm, table_hbm, out_hbm, table_sh, idx0, idx1,
                      rows0, rows1, sem_i0, sem_i1, sem_g, sem_o0, sem_o1):
        idx_v = (idx0, idx1)
        rows_v = (rows0, rows1)
        sem_i = (sem_i0, sem_i1)
        sem_o = (sem_o0, sem_o1)
        sid = lax.axis_index("s")
        wid = sid * nc + lax.axis_index("c")
        blk0 = wid * blocks_w

        start = jnp.minimum(sid * stripe, num_emb - stripe)
        pltpu.sync_copy(
            table_hbm.at[pl.ds(start, stripe)], table_sh.at[pl.ds(start, stripe)]
        )
        plsc.subcore_barrier()

        def idx_src(g):
            return idx_hbm.at[pl.ds(blk0 + g * _K, _K)]

        def do_group(g, b, wait_out):
            if wait_out:
                pltpu.make_async_copy(
                    rows_v[b], out_hbm.at[pl.ds(0, rows_per_group)], sem_o[b]
                ).wait()
            pltpu.make_async_copy(idx_src(0), idx_v[b], sem_i[b]).wait()
            copies = [
                pltpu.async_copy(
                    table_sh.at[idx_v[b].at[j]],
                    rows_v[b].at[pl.ds(j * _BLK, _BLK)],
                    sem_g,
                )
                for j in range(_K)
            ]
            for c in copies:
                c.wait()
            gn = jnp.minimum(g + _NBUF, groups - 1)
            pltpu.async_copy(idx_src(gn), idx_v[b], sem_i[b])
            pltpu.async_copy(
                rows_v[b],
                out_hbm.at[pl.ds((blk0 + g * _K) * _BLK, rows_per_group)],
                sem_o[b],
            )

        for b in range(_NBUF):
            pltpu.async_copy(idx_src(b), idx_v[b], sem_i[b])
        for b in range(_NBUF):
            do_group(jnp.int32(b), b, wait_out=False)

        def pair(p, carry):
            for b in range(_NBUF):
                do_group(p * _NBUF + b, b, wait_out=True)
            return carry

        lax.fori_loop(1, groups // _NBUF, pair, 0)

        for b in range(_NBUF):
            pltpu.make_async_copy(idx_src(0), idx_v[b], sem_i[b]).wait()
            pltpu.make_async_copy(
                rows_v[b], out_hbm.at[pl.ds(0, rows_per_group)], sem_o[b]
            ).wait()

    return gather_kernel


def kernel(sent_list, table):
    b, l = sent_list.shape
    n_total = b * l
    d = table.shape[1]
    idx2d = sent_list.reshape(n_total // _BLK, _BLK).astype(jnp.int32)
    out = _make_gather(n_total, table.shape[0], d)(idx2d, table.astype(jnp.float32))
    return out.reshape(b, l, d)
